# Initial kernel scaffold; baseline (speedup 1.0000x reference)
import jax
import jax.numpy as jnp
from jax.experimental import pallas as pl


def kernel(x, origin_feat, idx, wfc, bfc, wmu, bmu, wsig, bsig):
    raise NotImplementedError("write your pallas kernel here")


# speedup vs baseline: 5.8356x; 5.8356x over previous
import functools

import jax
import jax.numpy as jnp
from jax.experimental import pallas as pl
from jax.experimental.pallas import tpu as pltpu


def _stats_kernel(x_ref, idx_ref, wfc_ref, bfc_ref,
                  sum_ref, sumsq_ref, cnt_ref, *, n_total, chunk, tile_n):
    j = pl.program_id(1)

    @pl.when(j == 0)
    def _():
        sum_ref[...] = jnp.zeros_like(sum_ref)
        sumsq_ref[...] = jnp.zeros_like(sumsq_ref)
        cnt_ref[...] = jnp.zeros_like(cnt_ref)

    B = sum_ref.shape[1]
    y = jnp.dot(x_ref[...], wfc_ref[...],
                preferred_element_type=jnp.float32) + bfc_ref[...]

    gid = jax.lax.broadcasted_iota(jnp.int32, (B, tile_n), 0)
    sel = gid == idx_ref[...]
    if n_total % tile_n != 0:
        i = pl.program_id(0)
        col = (i * chunk + j) * tile_n + jax.lax.broadcasted_iota(
            jnp.int32, (1, tile_n), 1)
        sel = sel & (col < n_total)
    oh = jnp.where(sel, 1.0, 0.0)

    sum_ref[0] += jnp.dot(oh, y, preferred_element_type=jnp.float32)
    sumsq_ref[0] += jnp.dot(oh, y * y, preferred_element_type=jnp.float32)
    cnt_ref[0] += jnp.sum(oh, axis=1, keepdims=True)


def _finalize_kernel(sum_ref, sumsq_ref, cnt_ref, feat_ref, wms_ref, bms_ref,
                     tab_ref):
    outC = sum_ref.shape[2]
    nchunks = sum_ref.shape[0]
    s = sum_ref[0]
    ss = sumsq_ref[0]
    c = cnt_ref[0]
    for k in range(1, nchunks):
        s = s + sum_ref[k]
        ss = ss + sumsq_ref[k]
        c = c + cnt_ref[k]
    inv_c = 1.0 / jnp.maximum(c, 1.0)
    mean = s * inv_c
    var = jnp.maximum(ss * inv_c - mean * mean, 0.0)
    inv_std = jax.lax.rsqrt(var + 1e-14)
    musig = jnp.dot(feat_ref[...], wms_ref[...],
                    preferred_element_type=jnp.float32) + bms_ref[...]
    scale = musig[:, outC:] * inv_std
    shift = musig[:, :outC] - mean * scale
    tab_ref[:, :outC] = scale
    tab_ref[:, outC:] = shift


def _apply_kernel(x_ref, idx_ref, wfc_ref, bfc_ref, tab_ref, out_ref):
    tile_n, outC = out_ref.shape
    B = tab_ref.shape[0]

    y = jnp.dot(x_ref[...], wfc_ref[...],
                preferred_element_type=jnp.float32) + bfc_ref[...]

    gid = jax.lax.broadcasted_iota(jnp.int32, (B, tile_n), 0)
    oh = jnp.where(gid == idx_ref[...], 1.0, 0.0)
    dn = (((0,), (0,)), ((), ()))
    aff = jax.lax.dot_general(oh, tab_ref[...], dn,
                              preferred_element_type=jnp.float32)
    out_ref[...] = jnp.maximum(y * aff[:, :outC] + aff[:, outC:], 0.0)


def kernel(x, origin_feat, idx, wfc, bfc, wmu, bmu, wsig, bsig):
    N, inC = x.shape
    B, featC = origin_feat.shape
    outC = wfc.shape[1]

    tile_n = min(4096, N)
    grid_n = pl.cdiv(N, tile_n)
    ncores = 2 if grid_n % 2 == 0 else 1
    chunk = grid_n // ncores

    idx2d = idx.astype(jnp.int32).reshape(1, N)
    wms = jnp.concatenate([wmu, wsig], axis=1)
    bms = jnp.concatenate([bmu, bsig], axis=1)

    sums, sumsqs, cnts = pl.pallas_call(
        functools.partial(_stats_kernel, n_total=N, chunk=chunk, tile_n=tile_n),
        out_shape=(jax.ShapeDtypeStruct((ncores, B, outC), jnp.float32),
                   jax.ShapeDtypeStruct((ncores, B, outC), jnp.float32),
                   jax.ShapeDtypeStruct((ncores, B, 1), jnp.float32)),
        grid=(ncores, chunk),
        in_specs=[
            pl.BlockSpec((tile_n, inC), lambda i, j: (i * chunk + j, 0)),
            pl.BlockSpec((1, tile_n), lambda i, j: (0, i * chunk + j)),
            pl.BlockSpec((inC, outC), lambda i, j: (0, 0)),
            pl.BlockSpec((1, outC), lambda i, j: (0, 0)),
        ],
        out_specs=[
            pl.BlockSpec((1, B, outC), lambda i, j: (i, 0, 0)),
            pl.BlockSpec((1, B, outC), lambda i, j: (i, 0, 0)),
            pl.BlockSpec((1, B, 1), lambda i, j: (i, 0, 0)),
        ],
        compiler_params=pltpu.CompilerParams(
            dimension_semantics=("parallel", "arbitrary"),
            vmem_limit_bytes=64 * 1024 * 1024),
    )(x, idx2d, wfc, bfc)

    tab = pl.pallas_call(
        _finalize_kernel,
        out_shape=jax.ShapeDtypeStruct((B, 2 * outC), jnp.float32),
    )(sums, sumsqs, cnts, origin_feat, wms, bms)

    out = pl.pallas_call(
        _apply_kernel,
        out_shape=jax.ShapeDtypeStruct((N, outC), jnp.float32),
        grid=(grid_n,),
        in_specs=[
            pl.BlockSpec((tile_n, inC), lambda i: (i, 0)),
            pl.BlockSpec((1, tile_n), lambda i: (0, i)),
            pl.BlockSpec((inC, outC), lambda i: (0, 0)),
            pl.BlockSpec((1, outC), lambda i: (0, 0)),
            pl.BlockSpec((B, 2 * outC), lambda i: (0, 0)),
        ],
        out_specs=pl.BlockSpec((tile_n, outC), lambda i: (i, 0)),
        compiler_params=pltpu.CompilerParams(
            dimension_semantics=("parallel",),
            vmem_limit_bytes=64 * 1024 * 1024),
    )(x, idx2d, wfc, bfc, tab)

    return out
